# Initial kernel scaffold; baseline (speedup 1.0000x reference)
import jax
import jax.numpy as jnp
from jax.experimental import pallas as pl


def kernel(x, edge_index, W1, b1, W2, b2, W3, b3):
    raise NotImplementedError("write your pallas kernel here")


# speedup vs baseline: 13.0351x; 13.0351x over previous
import functools

import jax
import jax.numpy as jnp
from jax import lax
from jax.experimental import pallas as pl
from jax.experimental.pallas import tpu as pltpu
from jax.experimental.pallas import tpu_sc as plsc

N = 10000
D = 128
E = 320000

NC = 2
NS = 16
NW = NC * NS

C = 128
EDGES_MAIN = (E // NW // C) * C
CHUNKS = EDGES_MAIN // C
TAIL = E // NW - EDGES_MAIN
TAIL_BASE = EDGES_MAIN * NW

ROWS_A = 632
ROWS_B = N - 15 * ROWS_A

_mesh = plsc.VectorSubcoreMesh(core_axis_name="c", subcore_axis_name="s")


def _f32(*shape):
    return jax.ShapeDtypeStruct(shape, jnp.float32)


@functools.partial(
    pl.kernel,
    out_type=(_f32(N, D), _f32(N, D)),
    mesh=_mesh,
    scratch_types=[
        pltpu.VMEM((C,), jnp.int32),
        pltpu.VMEM((TAIL,), jnp.int32),
        pltpu.VMEM((C, D), jnp.float32),
        pltpu.VMEM((TAIL, D), jnp.float32),
        pltpu.VMEM_SHARED((N, D), jnp.float32),
    ],
)
def _degree_kernel(dst_hbm, ones_hbm, zeros_hbm, d0_hbm, d1_hbm,
                   dstv, dstv_t, ones_v, ones_t, acc):
    cid = lax.axis_index("c")
    sid = lax.axis_index("s")
    wid = cid * NS + sid

    pltpu.sync_copy(ones_hbm, ones_v)
    pltpu.sync_copy(ones_hbm.at[pl.ds(0, TAIL)], ones_t)

    @pl.when(sid < NS - 1)
    def _():
        pltpu.sync_copy(zeros_hbm, acc.at[pl.ds(sid * ROWS_A, ROWS_A)])

    @pl.when(sid == NS - 1)
    def _():
        pltpu.sync_copy(zeros_hbm.at[pl.ds(0, ROWS_B)],
                        acc.at[pl.ds(sid * ROWS_A, ROWS_B)])

    plsc.subcore_barrier()

    base = wid * EDGES_MAIN

    @pl.loop(0, CHUNKS)
    def _(i):
        pltpu.sync_copy(dst_hbm.at[pl.ds(base + i * C, C)], dstv)
        pltpu.sync_copy(ones_v, acc.at[dstv], add=True)

    pltpu.sync_copy(dst_hbm.at[pl.ds(TAIL_BASE + wid * TAIL, TAIL)], dstv_t)
    pltpu.sync_copy(ones_t, acc.at[dstv_t], add=True)

    plsc.subcore_barrier()

    @pl.when(sid < NS - 1)
    def _():
        @pl.when(cid == 0)
        def _():
            pltpu.sync_copy(acc.at[pl.ds(sid * ROWS_A, ROWS_A)],
                            d0_hbm.at[pl.ds(sid * ROWS_A, ROWS_A)])

        @pl.when(cid == 1)
        def _():
            pltpu.sync_copy(acc.at[pl.ds(sid * ROWS_A, ROWS_A)],
                            d1_hbm.at[pl.ds(sid * ROWS_A, ROWS_A)])

    @pl.when(sid == NS - 1)
    def _():
        @pl.when(cid == 0)
        def _():
            pltpu.sync_copy(acc.at[pl.ds(sid * ROWS_A, ROWS_B)],
                            d0_hbm.at[pl.ds(sid * ROWS_A, ROWS_B)])

        @pl.when(cid == 1)
        def _():
            pltpu.sync_copy(acc.at[pl.ds(sid * ROWS_A, ROWS_B)],
                            d1_hbm.at[pl.ds(sid * ROWS_A, ROWS_B)])


@functools.partial(
    pl.kernel,
    out_type=(_f32(N, D), _f32(N, D)),
    mesh=_mesh,
    scratch_types=[
        pltpu.VMEM((C,), jnp.int32),
        pltpu.VMEM((C,), jnp.int32),
        pltpu.VMEM((TAIL,), jnp.int32),
        pltpu.VMEM((TAIL,), jnp.int32),
        pltpu.VMEM((C, D), jnp.float32),
        pltpu.VMEM((TAIL, D), jnp.float32),
        pltpu.VMEM_SHARED((N, D), jnp.float32),
        pltpu.SemaphoreType.DMA,
    ],
)
def _propagate_kernel(g_hbm, src_hbm, dst_hbm, zeros_hbm, p0_hbm, p1_hbm,
                      srcv, dstv, srcv_t, dstv_t, rows, rows_t, acc, sem):
    cid = lax.axis_index("c")
    sid = lax.axis_index("s")
    wid = cid * NS + sid

    @pl.when(sid < NS - 1)
    def _():
        pltpu.sync_copy(zeros_hbm, acc.at[pl.ds(sid * ROWS_A, ROWS_A)])

    @pl.when(sid == NS - 1)
    def _():
        pltpu.sync_copy(zeros_hbm.at[pl.ds(0, ROWS_B)],
                        acc.at[pl.ds(sid * ROWS_A, ROWS_B)])

    plsc.subcore_barrier()

    base = wid * EDGES_MAIN

    @pl.loop(0, CHUNKS)
    def _(i):
        off = base + i * C
        pltpu.sync_copy(src_hbm.at[pl.ds(off, C)], srcv)
        pltpu.sync_copy(dst_hbm.at[pl.ds(off, C)], dstv)
        pltpu.async_copy(g_hbm.at[srcv], rows, sem).wait()
        pltpu.sync_copy(rows, acc.at[dstv], add=True)

    toff = TAIL_BASE + wid * TAIL
    pltpu.sync_copy(src_hbm.at[pl.ds(toff, TAIL)], srcv_t)
    pltpu.sync_copy(dst_hbm.at[pl.ds(toff, TAIL)], dstv_t)
    pltpu.async_copy(g_hbm.at[srcv_t], rows_t, sem).wait()
    pltpu.sync_copy(rows_t, acc.at[dstv_t], add=True)

    plsc.subcore_barrier()

    @pl.when(sid < NS - 1)
    def _():
        @pl.when(cid == 0)
        def _():
            pltpu.sync_copy(acc.at[pl.ds(sid * ROWS_A, ROWS_A)],
                            p0_hbm.at[pl.ds(sid * ROWS_A, ROWS_A)])

        @pl.when(cid == 1)
        def _():
            pltpu.sync_copy(acc.at[pl.ds(sid * ROWS_A, ROWS_A)],
                            p1_hbm.at[pl.ds(sid * ROWS_A, ROWS_A)])

    @pl.when(sid == NS - 1)
    def _():
        @pl.when(cid == 0)
        def _():
            pltpu.sync_copy(acc.at[pl.ds(sid * ROWS_A, ROWS_B)],
                            p0_hbm.at[pl.ds(sid * ROWS_A, ROWS_B)])

        @pl.when(cid == 1)
        def _():
            pltpu.sync_copy(acc.at[pl.ds(sid * ROWS_A, ROWS_B)],
                            p1_hbm.at[pl.ds(sid * ROWS_A, ROWS_B)])


R = 1000
GRID = N // R


def _dis_block(d0, d1):
    deg = d0[:, 0:1] + d1[:, 0:1] + 1.0
    return lax.rsqrt(deg)


def _pre_body(x_ref, w_ref, d0_ref, d1_ref, h_ref, g_ref):
    dis = _dis_block(d0_ref[...], d1_ref[...])
    h = jnp.dot(x_ref[...], w_ref[...], preferred_element_type=jnp.float32)
    h_ref[...] = h
    g_ref[...] = h * dis


def _mid_body(p0_ref, p1_ref, hp_ref, d0_ref, d1_ref, w_ref, b_ref,
              h_ref, g_ref):
    dis = _dis_block(d0_ref[...], d1_ref[...])
    out = dis * (p0_ref[...] + p1_ref[...]) + (dis * dis) * hp_ref[...] \
        + b_ref[...]
    t = jnp.maximum(out, 0.0)
    h = jnp.dot(t, w_ref[...], preferred_element_type=jnp.float32)
    h_ref[...] = h
    g_ref[...] = h * dis


def _post_body(p0_ref, p1_ref, hp_ref, d0_ref, d1_ref, b_ref, o_ref):
    dis = _dis_block(d0_ref[...], d1_ref[...])
    o_ref[...] = dis * (p0_ref[...] + p1_ref[...]) \
        + (dis * dis) * hp_ref[...] + b_ref[...]


_row_spec = pl.BlockSpec((R, D), lambda i: (i, 0))
_deg_spec = pl.BlockSpec((R, D), lambda i: (i, 0))
_w_spec = pl.BlockSpec((D, D), lambda i: (0, 0))
_b_spec = pl.BlockSpec((1, D), lambda i: (0, 0))

_pre_call = pl.pallas_call(
    _pre_body,
    grid=(GRID,),
    in_specs=[_row_spec, _w_spec, _deg_spec, _deg_spec],
    out_specs=[_row_spec, _row_spec],
    out_shape=(_f32(N, D), _f32(N, D)),
)

_mid_call = pl.pallas_call(
    _mid_body,
    grid=(GRID,),
    in_specs=[_row_spec, _row_spec, _row_spec, _deg_spec, _deg_spec,
              _w_spec, _b_spec],
    out_specs=[_row_spec, _row_spec],
    out_shape=(_f32(N, D), _f32(N, D)),
)

_post_call = pl.pallas_call(
    _post_body,
    grid=(GRID,),
    in_specs=[_row_spec, _row_spec, _row_spec, _deg_spec, _deg_spec, _b_spec],
    out_specs=_row_spec,
    out_shape=_f32(N, D),
)


def kernel(x, edge_index, W1, b1, W2, b2, W3, b3):
    src = edge_index[0].astype(jnp.int32)
    dst = edge_index[1].astype(jnp.int32)

    zeros_row = jnp.zeros((ROWS_A, D), jnp.float32)
    ones_row = jnp.ones((C, D), jnp.float32)

    d0, d1 = _degree_kernel(dst, ones_row, zeros_row)

    h1, g1 = _pre_call(x, W1, d0, d1)
    p0, p1 = _propagate_kernel(g1, src, dst, zeros_row)
    h2, g2 = _mid_call(p0, p1, h1, d0, d1, W2, b1.reshape(1, D))
    p0, p1 = _propagate_kernel(g2, src, dst, zeros_row)
    h3, g3 = _mid_call(p0, p1, h2, d0, d1, W3, b2.reshape(1, D))
    p0, p1 = _propagate_kernel(g3, src, dst, zeros_row)
    out = _post_call(p0, p1, h3, d0, d1, b3.reshape(1, D))
    return out


# speedup vs baseline: 23.2405x; 1.7829x over previous
import functools

import jax
import jax.numpy as jnp
from jax import lax
from jax.experimental import pallas as pl
from jax.experimental.pallas import tpu as pltpu
from jax.experimental.pallas import tpu_sc as plsc

N = 10000
D = 128
E = 320000

NC = 2
NS = 16
NW = NC * NS

C = 120
CHUNKS = 84
EPW = C * CHUNKS
E2 = EPW * NW
PAD = E2 - E
NA = N + 8

ROWS_A = 632
ROWS_B = N - 15 * ROWS_A

_mesh = plsc.VectorSubcoreMesh(core_axis_name="c", subcore_axis_name="s")


def _f32(*shape):
    return jax.ShapeDtypeStruct(shape, jnp.float32)


def _zero_acc(sid, zeros_hbm, acc):
    @pl.when(sid < NS - 1)
    def _():
        pltpu.sync_copy(zeros_hbm, acc.at[pl.ds(sid * ROWS_A, ROWS_A)])

    @pl.when(sid == NS - 1)
    def _():
        pltpu.sync_copy(zeros_hbm.at[pl.ds(0, ROWS_B)],
                        acc.at[pl.ds(sid * ROWS_A, ROWS_B)])


def _writeback(cid, sid, acc, p0_hbm, p1_hbm):
    @pl.when(sid < NS - 1)
    def _():
        @pl.when(cid == 0)
        def _():
            pltpu.sync_copy(acc.at[pl.ds(sid * ROWS_A, ROWS_A)],
                            p0_hbm.at[pl.ds(sid * ROWS_A, ROWS_A)])

        @pl.when(cid == 1)
        def _():
            pltpu.sync_copy(acc.at[pl.ds(sid * ROWS_A, ROWS_A)],
                            p1_hbm.at[pl.ds(sid * ROWS_A, ROWS_A)])

    @pl.when(sid == NS - 1)
    def _():
        @pl.when(cid == 0)
        def _():
            pltpu.sync_copy(acc.at[pl.ds(sid * ROWS_A, ROWS_B)],
                            p0_hbm.at[pl.ds(sid * ROWS_A, ROWS_B)])

        @pl.when(cid == 1)
        def _():
            pltpu.sync_copy(acc.at[pl.ds(sid * ROWS_A, ROWS_B)],
                            p1_hbm.at[pl.ds(sid * ROWS_A, ROWS_B)])


@functools.partial(
    pl.kernel,
    out_type=(_f32(N, D), _f32(N, D)),
    mesh=_mesh,
    scratch_types=[
        pltpu.VMEM((C,), jnp.int32),
        pltpu.VMEM((C,), jnp.int32),
        pltpu.VMEM((C,), jnp.int32),
        pltpu.VMEM((C, D), jnp.float32),
        pltpu.VMEM_SHARED((NA, D), jnp.float32),
        pltpu.SemaphoreType.DMA,
        pltpu.SemaphoreType.DMA,
        pltpu.SemaphoreType.DMA,
        pltpu.SemaphoreType.DMA,
        pltpu.SemaphoreType.DMA,
        pltpu.SemaphoreType.DMA,
    ],
)
def _degree_kernel(dst_hbm, ones_hbm, zeros_hbm, d0_hbm, d1_hbm,
                   dv0, dv1, dv2, ones_v, acc,
                   li0, li1, li2, s0, s1, s2):
    cid = lax.axis_index("c")
    sid = lax.axis_index("s")
    wid = cid * NS + sid
    base = wid * EPW

    pltpu.sync_copy(ones_hbm, ones_v)
    _zero_acc(sid, zeros_hbm, acc)
    plsc.subcore_barrier()

    dvs = (dv0, dv1, dv2)
    lis = (li0, li1, li2)
    sss = (s0, s1, s2)

    def idx_src(c):
        return dst_hbm.at[pl.ds(base + c * C, C)]

    for p in range(3):
        pltpu.async_copy(idx_src(p), dvs[p], lis[p])

    @pl.loop(0, CHUNKS, step=3)
    def _(j):
        for p in range(3):
            pltpu.make_async_copy(idx_src(j + p), dvs[p], lis[p]).wait()
            pltpu.async_copy(ones_v, acc.at[dvs[p]], sss[p], add=True)
        for p in range(3):
            c2 = jnp.minimum(j + 3 + p, CHUNKS - 1)
            pltpu.make_async_copy(ones_v, acc.at[dvs[p]], sss[p]).wait()
            pltpu.async_copy(idx_src(c2), dvs[p], lis[p])

    for p in range(3):
        pltpu.make_async_copy(idx_src(0), dvs[p], lis[p]).wait()

    plsc.subcore_barrier()
    _writeback(cid, sid, acc, d0_hbm, d1_hbm)


@functools.partial(
    pl.kernel,
    out_type=(_f32(N, D), _f32(N, D)),
    mesh=_mesh,
    scratch_types=[
        pltpu.VMEM((C,), jnp.int32),
        pltpu.VMEM((C,), jnp.int32),
        pltpu.VMEM((C,), jnp.int32),
        pltpu.VMEM((C,), jnp.int32),
        pltpu.VMEM((C,), jnp.int32),
        pltpu.VMEM((C,), jnp.int32),
        pltpu.VMEM((C, D), jnp.float32),
        pltpu.VMEM((C, D), jnp.float32),
        pltpu.VMEM((C, D), jnp.float32),
        pltpu.VMEM_SHARED((NA, D), jnp.float32),
        pltpu.SemaphoreType.DMA,
        pltpu.SemaphoreType.DMA,
        pltpu.SemaphoreType.DMA,
        pltpu.SemaphoreType.DMA,
        pltpu.SemaphoreType.DMA,
        pltpu.SemaphoreType.DMA,
        pltpu.SemaphoreType.DMA,
        pltpu.SemaphoreType.DMA,
        pltpu.SemaphoreType.DMA,
    ],
)
def _propagate_kernel(g_hbm, src_hbm, dst_hbm, zeros_hbm, p0_hbm, p1_hbm,
                      sv0, sv1, sv2, dv0, dv1, dv2, rows0, rows1, rows2, acc,
                      li0, li1, li2, g0, g1, g2, s0, s1, s2):
    cid = lax.axis_index("c")
    sid = lax.axis_index("s")
    wid = cid * NS + sid
    base = wid * EPW

    svs = (sv0, sv1, sv2)
    dvs = (dv0, dv1, dv2)
    rws = (rows0, rows1, rows2)
    lis = (li0, li1, li2)
    ggs = (g0, g1, g2)
    sss = (s0, s1, s2)

    def sidx(c):
        return src_hbm.at[pl.ds(base + c * C, C)]

    def didx(c):
        return dst_hbm.at[pl.ds(base + c * C, C)]

    _zero_acc(sid, zeros_hbm, acc)

    for p in range(3):
        pltpu.async_copy(sidx(p), svs[p], lis[p])
        pltpu.async_copy(didx(p), dvs[p], lis[p])

    plsc.subcore_barrier()

    for p in range(3):
        pltpu.make_async_copy(sidx(p), svs[p], lis[p]).wait()
        pltpu.make_async_copy(didx(p), dvs[p], lis[p]).wait()
        pltpu.async_copy(g_hbm.at[svs[p]], rws[p], ggs[p])

    @pl.loop(0, CHUNKS, step=3)
    def _(j):
        for p in range(3):
            pltpu.make_async_copy(g_hbm.at[svs[p]], rws[p], ggs[p]).wait()
            pltpu.async_copy(rws[p], acc.at[dvs[p]], sss[p], add=True)
        for p in range(3):
            c2 = jnp.minimum(j + 3 + p, CHUNKS - 1)
            pltpu.make_async_copy(rws[p], acc.at[dvs[p]], sss[p]).wait()
            pltpu.async_copy(sidx(c2), svs[p], lis[p])
            pltpu.async_copy(didx(c2), dvs[p], lis[p])
            pltpu.make_async_copy(sidx(c2), svs[p], lis[p]).wait()
            pltpu.make_async_copy(didx(c2), dvs[p], lis[p]).wait()
            pltpu.async_copy(g_hbm.at[svs[p]], rws[p], ggs[p])

    for p in range(3):
        pltpu.make_async_copy(g_hbm.at[svs[p]], rws[p], ggs[p]).wait()

    plsc.subcore_barrier()
    _writeback(cid, sid, acc, p0_hbm, p1_hbm)


R = 1000
GRID = N // R


def _dis_block(d0, d1):
    deg = d0[:, 0:1] + d1[:, 0:1] + 1.0
    return lax.rsqrt(deg)


def _pre_body(x_ref, w_ref, d0_ref, d1_ref, h_ref, g_ref):
    dis = _dis_block(d0_ref[...], d1_ref[...])
    h = jnp.dot(x_ref[...], w_ref[...], preferred_element_type=jnp.float32)
    h_ref[...] = h
    g_ref[...] = h * dis


def _mid_body(p0_ref, p1_ref, hp_ref, d0_ref, d1_ref, w_ref, b_ref,
              h_ref, g_ref):
    dis = _dis_block(d0_ref[...], d1_ref[...])
    out = dis * (p0_ref[...] + p1_ref[...]) + (dis * dis) * hp_ref[...] \
        + b_ref[...]
    t = jnp.maximum(out, 0.0)
    h = jnp.dot(t, w_ref[...], preferred_element_type=jnp.float32)
    h_ref[...] = h
    g_ref[...] = h * dis


def _post_body(p0_ref, p1_ref, hp_ref, d0_ref, d1_ref, b_ref, o_ref):
    dis = _dis_block(d0_ref[...], d1_ref[...])
    o_ref[...] = dis * (p0_ref[...] + p1_ref[...]) \
        + (dis * dis) * hp_ref[...] + b_ref[...]


_row_spec = pl.BlockSpec((R, D), lambda i: (i, 0))
_deg_spec = pl.BlockSpec((R, D), lambda i: (i, 0))
_w_spec = pl.BlockSpec((D, D), lambda i: (0, 0))
_b_spec = pl.BlockSpec((1, D), lambda i: (0, 0))

_pre_call = pl.pallas_call(
    _pre_body,
    grid=(GRID,),
    in_specs=[_row_spec, _w_spec, _deg_spec, _deg_spec],
    out_specs=[_row_spec, _row_spec],
    out_shape=(_f32(N, D), _f32(N, D)),
)

_mid_call = pl.pallas_call(
    _mid_body,
    grid=(GRID,),
    in_specs=[_row_spec, _row_spec, _row_spec, _deg_spec, _deg_spec,
              _w_spec, _b_spec],
    out_specs=[_row_spec, _row_spec],
    out_shape=(_f32(N, D), _f32(N, D)),
)

_post_call = pl.pallas_call(
    _post_body,
    grid=(GRID,),
    in_specs=[_row_spec, _row_spec, _row_spec, _deg_spec, _deg_spec, _b_spec],
    out_specs=_row_spec,
    out_shape=_f32(N, D),
)


def kernel(x, edge_index, W1, b1, W2, b2, W3, b3):
    src = edge_index[0].astype(jnp.int32)
    dst = edge_index[1].astype(jnp.int32)

    pad_src = (jnp.arange(PAD, dtype=jnp.int32) * 97) % N
    pad_dst = N + (jnp.arange(PAD, dtype=jnp.int32) % 8)
    src = jnp.concatenate([src, pad_src])
    dst = jnp.concatenate([dst, pad_dst])

    zeros_row = jnp.zeros((ROWS_A, D), jnp.float32)
    ones_row = jnp.ones((C, D), jnp.float32)

    d0, d1 = _degree_kernel(dst, ones_row, zeros_row)

    h1, g1 = _pre_call(x, W1, d0, d1)
    p0, p1 = _propagate_kernel(g1, src, dst, zeros_row)
    h2, g2 = _mid_call(p0, p1, h1, d0, d1, W2, b1.reshape(1, D))
    p0, p1 = _propagate_kernel(g2, src, dst, zeros_row)
    h3, g3 = _mid_call(p0, p1, h2, d0, d1, W3, b2.reshape(1, D))
    p0, p1 = _propagate_kernel(g3, src, dst, zeros_row)
    out = _post_call(p0, p1, h3, d0, d1, b3.reshape(1, D))
    return out
